# baseline (device time: 447402 ns/iter reference)
import jax
import jax.numpy as jnp
import numpy as np
from jax import lax
from jax.experimental import pallas as pl
from jax.experimental.pallas import tpu as pltpu

N_DEV = 4
B = 2
SQ_LOCAL = 512
SQ = N_DEV * SQ_LOCAL
D_MODEL = 1024
H_LOCAL = 8
D_HEAD = 128
QBLK = 512
SCALE = 0.08838834764831843

_CompilerParams = getattr(pltpu, "CompilerParams", None) or getattr(
    pltpu, "TPUCompilerParams"
)


def _rope_tables():
    inv = 1.0 / (10000.0 ** (np.arange(0, D_HEAD, 2) / D_HEAD))
    pos = np.arange(SQ)[:, None] * inv[None, :]
    cos = np.repeat(np.cos(pos), 2, axis=-1).astype(np.float32)
    sin = np.repeat(np.sin(pos), 2, axis=-1).astype(np.float32)
    rot = np.zeros((D_HEAD, D_HEAD), np.float32)
    for k2 in range(0, D_HEAD, 2):
        rot[k2 + 1, k2] = -1.0
        rot[k2, k2 + 1] = 1.0
    return cos, sin, rot


_COS_NP, _SIN_NP, _ROT_NP = _rope_tables()


def _ag_body(x_ref, out_ref, comm_ref, send_sems, recv_sems):
    my = lax.axis_index("i")
    left = (my - 1) % N_DEV
    right = (my + 1) % N_DEV

    barrier = pltpu.get_barrier_semaphore()
    for nbr in (left, right):
        pl.semaphore_signal(
            barrier, inc=1, device_id=(nbr,), device_id_type=pl.DeviceIdType.MESH
        )
    pl.semaphore_wait(barrier, 2)

    out_ref[:, pl.ds(my * SQ_LOCAL, SQ_LOCAL), :] = x_ref[...]
    comm_ref[0, :, :, :] = x_ref[...]

    for h in range(N_DEV - 1):
        rdma = pltpu.make_async_remote_copy(
            src_ref=comm_ref.at[h],
            dst_ref=comm_ref.at[h + 1],
            send_sem=send_sems.at[h],
            recv_sem=recv_sems.at[h],
            device_id=(right,),
            device_id_type=pl.DeviceIdType.MESH,
        )
        rdma.start()
        rdma.wait()
        origin = (my - h - 1) % N_DEV
        out_ref[:, pl.ds(origin * SQ_LOCAL, SQ_LOCAL), :] = comm_ref[h + 1]


def _all_gather(x_bf):
    return pl.pallas_call(
        _ag_body,
        out_shape=jax.ShapeDtypeStruct((B, SQ, D_MODEL), jnp.bfloat16),
        in_specs=[pl.BlockSpec(memory_space=pltpu.VMEM)],
        out_specs=pl.BlockSpec(memory_space=pltpu.VMEM),
        scratch_shapes=[
            pltpu.VMEM((N_DEV, B, SQ_LOCAL, D_MODEL), jnp.bfloat16),
            pltpu.SemaphoreType.DMA((N_DEV - 1,)),
            pltpu.SemaphoreType.DMA((N_DEV - 1,)),
        ],
        compiler_params=_CompilerParams(collective_id=0),
    )(x_bf)


def _attn_body(x_ref, wq_ref, wk_ref, wv_ref, wo_ref, cos_ref, sin_ref,
               rot_ref, out_ref):
    h = pl.program_id(1)
    xb = x_ref[0]
    cos = cos_ref[...]
    sin = sin_ref[...]
    rot = rot_ref[...]

    q = jnp.dot(xb, wq_ref[...], preferred_element_type=jnp.float32)
    k = jnp.dot(xb, wk_ref[...], preferred_element_type=jnp.float32)
    v = jnp.dot(xb, wv_ref[...], preferred_element_type=jnp.float32)
    v = v.astype(jnp.bfloat16)

    def rope(t):
        return t * cos + jnp.dot(t, rot, preferred_element_type=jnp.float32) * sin

    qr = (rope(q) * SCALE).astype(jnp.bfloat16)
    kr = rope(k).astype(jnp.bfloat16)

    for i in range(SQ // QBLK):
        sl = slice(i * QBLK, (i + 1) * QBLK)
        s = lax.dot_general(
            qr[sl], kr, (((1,), (1,)), ((), ())),
            preferred_element_type=jnp.float32,
        )
        m = jnp.max(s, axis=-1, keepdims=True)
        p = jnp.exp(s - m)
        denom = jnp.sum(p, axis=-1, keepdims=True)
        pw = (p / denom).astype(jnp.bfloat16)
        ctx = jnp.dot(pw, v, preferred_element_type=jnp.float32)
        delta = jnp.dot(
            ctx.astype(jnp.bfloat16), wo_ref[...],
            preferred_element_type=jnp.float32,
        )

        @pl.when(h == 0)
        def _():
            out_ref[0, sl, :] = delta

        @pl.when(h != 0)
        def _():
            out_ref[0, sl, :] = out_ref[0, sl, :] + delta


def _attention(x_full, wq, wk, wv, wo, cos, sin, rot):
    return pl.pallas_call(
        _attn_body,
        grid=(B, H_LOCAL),
        in_specs=[
            pl.BlockSpec((1, SQ, D_MODEL), lambda b, h: (b, 0, 0)),
            pl.BlockSpec((D_MODEL, D_HEAD), lambda b, h: (0, h)),
            pl.BlockSpec((D_MODEL, D_HEAD), lambda b, h: (0, h)),
            pl.BlockSpec((D_MODEL, D_HEAD), lambda b, h: (0, h)),
            pl.BlockSpec((D_HEAD, D_MODEL), lambda b, h: (h, 0)),
            pl.BlockSpec((SQ, D_HEAD), lambda b, h: (0, 0)),
            pl.BlockSpec((SQ, D_HEAD), lambda b, h: (0, 0)),
            pl.BlockSpec((D_HEAD, D_HEAD), lambda b, h: (0, 0)),
        ],
        out_specs=pl.BlockSpec((1, SQ, D_MODEL), lambda b, h: (b, 0, 0)),
        out_shape=jax.ShapeDtypeStruct((B, SQ, D_MODEL), jnp.float32),
    )(x_full, wq, wk, wv, wo, cos, sin, rot)


def _rs_body(p_ref, out_ref, send_buf, recv_buf, send_sems, recv_sems):
    my = lax.axis_index("i")
    left = (my - 1) % N_DEV
    right = (my + 1) % N_DEV

    barrier = pltpu.get_barrier_semaphore()
    for nbr in (left, right):
        pl.semaphore_signal(
            barrier, inc=1, device_id=(nbr,), device_id_type=pl.DeviceIdType.MESH
        )
    pl.semaphore_wait(barrier, 2)

    for s in range(N_DEV - 1):
        c_send = (my + N_DEV - 1 - s) % N_DEV
        chunk = p_ref[:, pl.ds(c_send * SQ_LOCAL, SQ_LOCAL), :]
        if s == 0:
            send_buf[...] = chunk
        else:
            send_buf[...] = chunk + recv_buf[s - 1]
        rdma = pltpu.make_async_remote_copy(
            src_ref=send_buf,
            dst_ref=recv_buf.at[s],
            send_sem=send_sems.at[s],
            recv_sem=recv_sems.at[s],
            device_id=(right,),
            device_id_type=pl.DeviceIdType.MESH,
        )
        rdma.start()
        rdma.wait()

    mine = p_ref[:, pl.ds(my * SQ_LOCAL, SQ_LOCAL), :].astype(jnp.float32)
    out_ref[...] = mine + recv_buf[N_DEV - 2].astype(jnp.float32)


def _reduce_scatter(partial_bf):
    return pl.pallas_call(
        _rs_body,
        out_shape=jax.ShapeDtypeStruct((B, SQ_LOCAL, D_MODEL), jnp.float32),
        in_specs=[pl.BlockSpec(memory_space=pltpu.VMEM)],
        out_specs=pl.BlockSpec(memory_space=pltpu.VMEM),
        scratch_shapes=[
            pltpu.VMEM((B, SQ_LOCAL, D_MODEL), jnp.bfloat16),
            pltpu.VMEM((N_DEV - 1, B, SQ_LOCAL, D_MODEL), jnp.bfloat16),
            pltpu.SemaphoreType.DMA((N_DEV - 1,)),
            pltpu.SemaphoreType.DMA((N_DEV - 1,)),
        ],
        compiler_params=_CompilerParams(collective_id=1),
    )(partial_bf)


def kernel(x, Wq, Wk, Wv, Wo):
    x_bf = x.astype(jnp.bfloat16)
    wq = Wq.astype(jnp.bfloat16)
    wk = Wk.astype(jnp.bfloat16)
    wv = Wv.astype(jnp.bfloat16)
    wo = Wo.astype(jnp.bfloat16)
    cos = jnp.asarray(_COS_NP)
    sin = jnp.asarray(_SIN_NP)
    rot = jnp.asarray(_ROT_NP)

    x_full = _all_gather(x_bf)
    partial = _attention(x_full, wq, wk, wv, wo, cos, sin, rot)
    out = _reduce_scatter(partial.astype(jnp.bfloat16))
    return out


# device time: 373627 ns/iter; 1.1975x vs baseline; 1.1975x over previous
import jax
import jax.numpy as jnp
import numpy as np
from jax import lax
from jax.experimental import pallas as pl
from jax.experimental.pallas import tpu as pltpu

N_DEV = 4
B = 2
SQ_LOCAL = 512
SQ = N_DEV * SQ_LOCAL
D_MODEL = 1024
H_LOCAL = 8
D_HEAD = 128
QBLK = 512
SCALE = 0.08838834764831843

_CompilerParams = getattr(pltpu, "CompilerParams", None) or getattr(
    pltpu, "TPUCompilerParams"
)


def _rope_tables():
    inv = 1.0 / (10000.0 ** (np.arange(0, D_HEAD, 2) / D_HEAD))
    pos = np.arange(SQ)[:, None] * inv[None, :]
    cos = np.repeat(np.cos(pos), 2, axis=-1).astype(np.float32)
    sin = np.repeat(np.sin(pos), 2, axis=-1).astype(np.float32)
    rot = np.zeros((D_HEAD, D_HEAD), np.float32)
    for k2 in range(0, D_HEAD, 2):
        rot[k2 + 1, k2] = -1.0
        rot[k2, k2 + 1] = 1.0
    return cos, sin, rot


_COS_NP, _SIN_NP, _ROT_NP = _rope_tables()


def _ag_body(x_ref, out_ref, comm_ref, send_sems, recv_sems):
    my = lax.axis_index("i")
    left = (my - 1) % N_DEV
    right = (my + 1) % N_DEV

    barrier = pltpu.get_barrier_semaphore()
    for nbr in (left, right):
        pl.semaphore_signal(
            barrier, inc=1, device_id=(nbr,), device_id_type=pl.DeviceIdType.MESH
        )
    pl.semaphore_wait(barrier, 2)

    out_ref[:, pl.ds(my * SQ_LOCAL, SQ_LOCAL), :] = x_ref[...]
    comm_ref[0, :, :, :] = x_ref[...]

    for h in range(N_DEV - 1):
        rdma = pltpu.make_async_remote_copy(
            src_ref=comm_ref.at[h],
            dst_ref=comm_ref.at[h + 1],
            send_sem=send_sems.at[h],
            recv_sem=recv_sems.at[h],
            device_id=(right,),
            device_id_type=pl.DeviceIdType.MESH,
        )
        rdma.start()
        rdma.wait()
        origin = (my - h - 1) % N_DEV
        out_ref[:, pl.ds(origin * SQ_LOCAL, SQ_LOCAL), :] = comm_ref[h + 1]


def _all_gather(x_bf):
    return pl.pallas_call(
        _ag_body,
        out_shape=jax.ShapeDtypeStruct((B, SQ, D_MODEL), jnp.bfloat16),
        in_specs=[pl.BlockSpec(memory_space=pltpu.VMEM)],
        out_specs=pl.BlockSpec(memory_space=pltpu.VMEM),
        scratch_shapes=[
            pltpu.VMEM((N_DEV, B, SQ_LOCAL, D_MODEL), jnp.bfloat16),
            pltpu.SemaphoreType.DMA((N_DEV - 1,)),
            pltpu.SemaphoreType.DMA((N_DEV - 1,)),
        ],
        compiler_params=_CompilerParams(collective_id=0),
    )(x_bf)


def _attn_body(x_ref, wq_ref, wk_ref, wv_ref, wo_ref, cos_ref, sin_ref,
               rot_ref, out_ref, qs_ref, ks_ref, vs_ref):
    b = pl.program_id(0)
    h = pl.program_id(1)

    @pl.when(h == 0)
    def _():
        xb = x_ref[b]
        cos = cos_ref[...]
        sin = sin_ref[...]
        rot = rot_ref[...]

        for w_ref, dst_ref, do_rope, scale in (
            (wq_ref, qs_ref, True, SCALE),
            (wk_ref, ks_ref, True, 1.0),
            (wv_ref, vs_ref, False, 1.0),
        ):
            for cc in range(0, D_MODEL, 512):
                t = jnp.dot(
                    xb, w_ref[:, cc:cc + 512],
                    preferred_element_type=jnp.float32,
                )
                for j in range(512 // D_HEAD):
                    hh = cc // D_HEAD + j
                    tcol = t[:, j * D_HEAD:(j + 1) * D_HEAD]
                    if do_rope:
                        tr = jnp.dot(
                            tcol.astype(jnp.bfloat16), rot,
                            preferred_element_type=jnp.float32,
                        )
                        tcol = (tcol * cos + tr * sin) * scale
                    dst_ref[hh] = tcol.astype(jnp.bfloat16)

    q = qs_ref[h]
    k = ks_ref[h]
    v = vs_ref[h]
    wo_h = wo_ref[pl.ds(h * D_HEAD, D_HEAD), :]

    for i in range(SQ // QBLK):
        sl = slice(i * QBLK, (i + 1) * QBLK)
        s = lax.dot_general(
            q[sl], k, (((1,), (1,)), ((), ())),
            preferred_element_type=jnp.float32,
        )
        p = jnp.exp(s)
        denom = jnp.sum(p, axis=-1, keepdims=True)
        ctx = jnp.dot(p.astype(jnp.bfloat16), v,
                      preferred_element_type=jnp.float32)
        ctx = ctx * (1.0 / denom)
        delta = jnp.dot(ctx.astype(jnp.bfloat16), wo_h,
                        preferred_element_type=jnp.float32)

        @pl.when(h == 0)
        def _():
            out_ref[b, sl, :] = delta

        @pl.when(h != 0)
        def _():
            out_ref[b, sl, :] = out_ref[b, sl, :] + delta


def _attention(x_full, wq, wk, wv, wo, cos, sin, rot):
    full = pl.BlockSpec(memory_space=pltpu.VMEM)
    return pl.pallas_call(
        _attn_body,
        grid=(B, H_LOCAL),
        in_specs=[full] * 8,
        out_specs=full,
        out_shape=jax.ShapeDtypeStruct((B, SQ, D_MODEL), jnp.float32),
        scratch_shapes=[
            pltpu.VMEM((H_LOCAL, SQ, D_HEAD), jnp.bfloat16),
            pltpu.VMEM((H_LOCAL, SQ, D_HEAD), jnp.bfloat16),
            pltpu.VMEM((H_LOCAL, SQ, D_HEAD), jnp.bfloat16),
        ],
        compiler_params=_CompilerParams(vmem_limit_bytes=60 * 1024 * 1024),
    )(x_full, wq, wk, wv, wo, cos, sin, rot)


def _rs_body(p_ref, out_ref, send_buf, recv_buf, send_sems, recv_sems):
    my = lax.axis_index("i")
    left = (my - 1) % N_DEV
    right = (my + 1) % N_DEV

    barrier = pltpu.get_barrier_semaphore()
    for nbr in (left, right):
        pl.semaphore_signal(
            barrier, inc=1, device_id=(nbr,), device_id_type=pl.DeviceIdType.MESH
        )
    pl.semaphore_wait(barrier, 2)

    for s in range(N_DEV - 1):
        c_send = (my + N_DEV - 1 - s) % N_DEV
        chunk = p_ref[:, pl.ds(c_send * SQ_LOCAL, SQ_LOCAL), :]
        if s == 0:
            send_buf[...] = chunk
        else:
            send_buf[...] = chunk + recv_buf[s - 1]
        rdma = pltpu.make_async_remote_copy(
            src_ref=send_buf,
            dst_ref=recv_buf.at[s],
            send_sem=send_sems.at[s],
            recv_sem=recv_sems.at[s],
            device_id=(right,),
            device_id_type=pl.DeviceIdType.MESH,
        )
        rdma.start()
        rdma.wait()

    mine = p_ref[:, pl.ds(my * SQ_LOCAL, SQ_LOCAL), :].astype(jnp.float32)
    out_ref[...] = mine + recv_buf[N_DEV - 2].astype(jnp.float32)


def _reduce_scatter(partial_bf):
    return pl.pallas_call(
        _rs_body,
        out_shape=jax.ShapeDtypeStruct((B, SQ_LOCAL, D_MODEL), jnp.float32),
        in_specs=[pl.BlockSpec(memory_space=pltpu.VMEM)],
        out_specs=pl.BlockSpec(memory_space=pltpu.VMEM),
        scratch_shapes=[
            pltpu.VMEM((B, SQ_LOCAL, D_MODEL), jnp.bfloat16),
            pltpu.VMEM((N_DEV - 1, B, SQ_LOCAL, D_MODEL), jnp.bfloat16),
            pltpu.SemaphoreType.DMA((N_DEV - 1,)),
            pltpu.SemaphoreType.DMA((N_DEV - 1,)),
        ],
        compiler_params=_CompilerParams(collective_id=1),
    )(partial_bf)


def kernel(x, Wq, Wk, Wv, Wo):
    x_bf = x.astype(jnp.bfloat16)
    wq = Wq.astype(jnp.bfloat16)
    wk = Wk.astype(jnp.bfloat16)
    wv = Wv.astype(jnp.bfloat16)
    wo = Wo.astype(jnp.bfloat16)
    cos = jnp.asarray(_COS_NP)
    sin = jnp.asarray(_SIN_NP)
    rot = jnp.asarray(_ROT_NP).astype(jnp.bfloat16)

    x_full = _all_gather(x_bf)
    partial = _attention(x_full, wq, wk, wv, wo, cos, sin, rot)
    out = _reduce_scatter(partial.astype(jnp.bfloat16))
    return out


# device time: 296488 ns/iter; 1.5090x vs baseline; 1.2602x over previous
import jax
import jax.numpy as jnp
import numpy as np
from jax import lax
from jax.experimental import pallas as pl
from jax.experimental.pallas import tpu as pltpu

N_DEV = 4
B = 2
SQ_LOCAL = 512
SQ = N_DEV * SQ_LOCAL
D_MODEL = 1024
H_LOCAL = 8
D_HEAD = 128
QBLK = 512
SCALE = 0.08838834764831843

_CompilerParams = getattr(pltpu, "CompilerParams", None) or getattr(
    pltpu, "TPUCompilerParams"
)


def _rope_tables():
    inv = 1.0 / (10000.0 ** (np.arange(0, D_HEAD, 2) / D_HEAD))
    pos = np.arange(SQ)[:, None] * inv[None, :]
    cos = np.repeat(np.cos(pos), 2, axis=-1).astype(np.float32)
    sin = np.repeat(np.sin(pos), 2, axis=-1).astype(np.float32)
    rot = np.zeros((D_HEAD, D_HEAD), np.float32)
    for k2 in range(0, D_HEAD, 2):
        rot[k2 + 1, k2] = -1.0
        rot[k2, k2 + 1] = 1.0
    return cos, sin, rot


_COS_NP, _SIN_NP, _ROT_NP = _rope_tables()


def _ag_body(x_ref, out_ref, comm_ref, send_sems, recv_sems):
    my = lax.axis_index("i")
    left = (my - 1) % N_DEV
    right = (my + 1) % N_DEV

    barrier = pltpu.get_barrier_semaphore()
    for nbr in (left, right):
        pl.semaphore_signal(
            barrier, inc=1, device_id=(nbr,), device_id_type=pl.DeviceIdType.MESH
        )
    pl.semaphore_wait(barrier, 2)

    out_ref[:, pl.ds(my * SQ_LOCAL, SQ_LOCAL), :] = x_ref[...]
    comm_ref[0, :, :, :] = x_ref[...]

    for h in range(N_DEV - 1):
        rdma = pltpu.make_async_remote_copy(
            src_ref=comm_ref.at[h],
            dst_ref=comm_ref.at[h + 1],
            send_sem=send_sems.at[h],
            recv_sem=recv_sems.at[h],
            device_id=(right,),
            device_id_type=pl.DeviceIdType.MESH,
        )
        rdma.start()
        rdma.wait()
        origin = (my - h - 1) % N_DEV
        out_ref[:, pl.ds(origin * SQ_LOCAL, SQ_LOCAL), :] = comm_ref[h + 1]


def _all_gather(x_bf):
    return pl.pallas_call(
        _ag_body,
        out_shape=jax.ShapeDtypeStruct((B, SQ, D_MODEL), jnp.bfloat16),
        in_specs=[pl.BlockSpec(memory_space=pltpu.VMEM)],
        out_specs=pl.BlockSpec(memory_space=pltpu.VMEM),
        scratch_shapes=[
            pltpu.VMEM((N_DEV, B, SQ_LOCAL, D_MODEL), jnp.bfloat16),
            pltpu.SemaphoreType.DMA((N_DEV - 1,)),
            pltpu.SemaphoreType.DMA((N_DEV - 1,)),
        ],
        compiler_params=_CompilerParams(collective_id=0),
    )(x_bf)


def _attn_body(x_ref, wq_ref, wk_ref, wv_ref, wo_ref, cos_ref, sin_ref,
               rot_ref, out_ref, qs_ref, ks_ref, vs_ref):
    b = pl.program_id(0)
    h = pl.program_id(1)

    @pl.when(h == 0)
    def _():
        xb = x_ref[b]
        cos = cos_ref[...]
        sin = sin_ref[...]
        rot = rot_ref[...]

        for w_ref, dst_ref, do_rope, scale in (
            (wq_ref, qs_ref, True, SCALE),
            (wk_ref, ks_ref, True, 1.0),
            (wv_ref, vs_ref, False, 1.0),
        ):
            for cc in range(0, D_MODEL, 512):
                t = jnp.dot(
                    xb, w_ref[:, cc:cc + 512],
                    preferred_element_type=jnp.float32,
                )
                for j in range(512 // D_HEAD):
                    hh = cc // D_HEAD + j
                    tcol = t[:, j * D_HEAD:(j + 1) * D_HEAD]
                    if do_rope:
                        tr = jnp.dot(
                            tcol.astype(jnp.bfloat16), rot,
                            preferred_element_type=jnp.float32,
                        )
                        tcol = (tcol * cos + tr * sin) * scale
                    dst_ref[hh] = tcol.astype(jnp.bfloat16)

    q = qs_ref[h]
    k = ks_ref[h]
    v = vs_ref[h]
    wo_h = wo_ref[pl.ds(h * D_HEAD, D_HEAD), :]

    for i in range(SQ // QBLK):
        sl = slice(i * QBLK, (i + 1) * QBLK)
        s = lax.dot_general(
            q[sl], k, (((1,), (1,)), ((), ())),
            preferred_element_type=jnp.float32,
        )
        p = jnp.exp(s)
        denom = jnp.sum(p, axis=-1, keepdims=True)
        ctx = jnp.dot(p.astype(jnp.bfloat16), v,
                      preferred_element_type=jnp.float32)
        ctx = ctx * (1.0 / denom)
        delta = jnp.dot(ctx.astype(jnp.bfloat16), wo_h,
                        preferred_element_type=jnp.float32)

        @pl.when(h == 0)
        def _():
            out_ref[b, sl, :] = delta

        @pl.when(h != 0)
        def _():
            out_ref[b, sl, :] = out_ref[b, sl, :] + delta


def _attention(x_full, wq, wk, wv, wo, cos, sin, rot):
    full = pl.BlockSpec(memory_space=pltpu.VMEM)
    return pl.pallas_call(
        _attn_body,
        grid=(B, H_LOCAL),
        in_specs=[full] * 8,
        out_specs=full,
        out_shape=jax.ShapeDtypeStruct((B, SQ, D_MODEL), jnp.float32),
        scratch_shapes=[
            pltpu.VMEM((H_LOCAL, SQ, D_HEAD), jnp.bfloat16),
            pltpu.VMEM((H_LOCAL, SQ, D_HEAD), jnp.bfloat16),
            pltpu.VMEM((H_LOCAL, SQ, D_HEAD), jnp.bfloat16),
        ],
        compiler_params=_CompilerParams(vmem_limit_bytes=60 * 1024 * 1024),
    )(x_full, wq, wk, wv, wo, cos, sin, rot)


def _attn_rs_body(x_ref, wq_ref, wk_ref, wv_ref, wo_ref, cos_ref, sin_ref,
                  rot_ref, out_ref, ks_ref, vs_ref, qc_ref, send_ref, recv_ref,
                  send_sems, recv_sems):
    my = lax.axis_index("i")
    left = (my - 1) % N_DEV
    right = (my + 1) % N_DEV

    barrier = pltpu.get_barrier_semaphore()
    for nbr in (left, right):
        pl.semaphore_signal(
            barrier, inc=1, device_id=(nbr,), device_id_type=pl.DeviceIdType.MESH
        )
    pl.semaphore_wait(barrier, 2)

    rot = rot_ref[...]

    for b in range(B):
        xb = x_ref[b]
        for w_ref, dst_ref, do_rope in (
            (wk_ref, ks_ref, True),
            (wv_ref, vs_ref, False),
        ):
            for cc in range(0, D_MODEL, 512):
                t = jnp.dot(xb, w_ref[:, cc:cc + 512],
                            preferred_element_type=jnp.float32)
                for j in range(512 // D_HEAD):
                    hh = cc // D_HEAD + j
                    tcol = t[:, j * D_HEAD:(j + 1) * D_HEAD]
                    if do_rope:
                        tr = jnp.dot(tcol.astype(jnp.bfloat16), rot,
                                     preferred_element_type=jnp.float32)
                        tcol = tcol * cos_ref[...] + tr * sin_ref[...]
                    dst_ref[b, hh] = tcol.astype(jnp.bfloat16)

    rdmas = [
        pltpu.make_async_remote_copy(
            src_ref=send_ref,
            dst_ref=recv_ref.at[s],
            send_sem=send_sems.at[s],
            recv_sem=recv_sems.at[s],
            device_id=(right,),
            device_id_type=pl.DeviceIdType.MESH,
        )
        for s in range(N_DEV - 1)
    ]

    for s in range(N_DEV):
        c = (my + N_DEV - 1 - s) % N_DEV
        row = pl.ds(c * SQ_LOCAL, SQ_LOCAL)
        cos_c = cos_ref[row, :]
        sin_c = sin_ref[row, :]
        accs = []
        for b in range(B):
            xc = x_ref[b, row, :]
            for cc in range(0, D_MODEL, 512):
                t = jnp.dot(xc, wq_ref[:, cc:cc + 512],
                            preferred_element_type=jnp.float32)
                for j in range(512 // D_HEAD):
                    hh = cc // D_HEAD + j
                    tcol = t[:, j * D_HEAD:(j + 1) * D_HEAD]
                    tr = jnp.dot(tcol.astype(jnp.bfloat16), rot,
                                 preferred_element_type=jnp.float32)
                    qc_ref[hh] = (
                        (tcol * cos_c + tr * sin_c) * SCALE
                    ).astype(jnp.bfloat16)
            def head_body(h, acc, b=b):
                sres = lax.dot_general(
                    qc_ref[h], ks_ref[b, h], (((1,), (1,)), ((), ())),
                    preferred_element_type=jnp.float32,
                )
                p = jnp.exp(sres)
                denom = jnp.sum(p, axis=-1, keepdims=True)
                ctx = jnp.dot(p.astype(jnp.bfloat16), vs_ref[b, h],
                              preferred_element_type=jnp.float32)
                ctx = ctx * (1.0 / denom)
                return acc + jnp.dot(
                    ctx.astype(jnp.bfloat16),
                    wo_ref[pl.ds(h * D_HEAD, D_HEAD), :],
                    preferred_element_type=jnp.float32,
                )

            acc = lax.fori_loop(
                0, H_LOCAL, head_body,
                jnp.zeros((SQ_LOCAL, D_MODEL), jnp.float32),
            )
            accs.append(acc)

        if s == 0:
            for b in range(B):
                send_ref[b] = accs[b].astype(jnp.bfloat16)
            rdmas[0].start()
        elif s < N_DEV - 1:
            rdmas[s - 1].wait_recv()
            rdmas[s - 1].wait_send()
            for b in range(B):
                send_ref[b] = (
                    accs[b] + recv_ref[s - 1, b].astype(jnp.float32)
                ).astype(jnp.bfloat16)
            rdmas[s].start()
        else:
            rdmas[s - 1].wait_recv()
            for b in range(B):
                out_ref[b] = accs[b] + recv_ref[s - 1, b].astype(jnp.float32)
            rdmas[s - 1].wait_send()


def _attn_rs(x_full, wq, wk, wv, wo, cos, sin, rot):
    full = pl.BlockSpec(memory_space=pltpu.VMEM)
    return pl.pallas_call(
        _attn_rs_body,
        out_shape=jax.ShapeDtypeStruct((B, SQ_LOCAL, D_MODEL), jnp.float32),
        in_specs=[full] * 8,
        out_specs=full,
        scratch_shapes=[
            pltpu.VMEM((B, H_LOCAL, SQ, D_HEAD), jnp.bfloat16),
            pltpu.VMEM((B, H_LOCAL, SQ, D_HEAD), jnp.bfloat16),
            pltpu.VMEM((H_LOCAL, SQ_LOCAL, D_HEAD), jnp.bfloat16),
            pltpu.VMEM((B, SQ_LOCAL, D_MODEL), jnp.bfloat16),
            pltpu.VMEM((N_DEV - 1, B, SQ_LOCAL, D_MODEL), jnp.bfloat16),
            pltpu.SemaphoreType.DMA((N_DEV - 1,)),
            pltpu.SemaphoreType.DMA((N_DEV - 1,)),
        ],
        compiler_params=_CompilerParams(
            collective_id=1, vmem_limit_bytes=63 * 1024 * 1024
        ),
    )(x_full, wq, wk, wv, wo, cos, sin, rot)


def _rs_body(p_ref, out_ref, send_buf, recv_buf, send_sems, recv_sems):
    my = lax.axis_index("i")
    left = (my - 1) % N_DEV
    right = (my + 1) % N_DEV

    barrier = pltpu.get_barrier_semaphore()
    for nbr in (left, right):
        pl.semaphore_signal(
            barrier, inc=1, device_id=(nbr,), device_id_type=pl.DeviceIdType.MESH
        )
    pl.semaphore_wait(barrier, 2)

    for s in range(N_DEV - 1):
        c_send = (my + N_DEV - 1 - s) % N_DEV
        chunk = p_ref[:, pl.ds(c_send * SQ_LOCAL, SQ_LOCAL), :]
        if s == 0:
            send_buf[...] = chunk
        else:
            send_buf[...] = chunk + recv_buf[s - 1]
        rdma = pltpu.make_async_remote_copy(
            src_ref=send_buf,
            dst_ref=recv_buf.at[s],
            send_sem=send_sems.at[s],
            recv_sem=recv_sems.at[s],
            device_id=(right,),
            device_id_type=pl.DeviceIdType.MESH,
        )
        rdma.start()
        rdma.wait()

    mine = p_ref[:, pl.ds(my * SQ_LOCAL, SQ_LOCAL), :].astype(jnp.float32)
    out_ref[...] = mine + recv_buf[N_DEV - 2].astype(jnp.float32)


def _reduce_scatter(partial_bf):
    return pl.pallas_call(
        _rs_body,
        out_shape=jax.ShapeDtypeStruct((B, SQ_LOCAL, D_MODEL), jnp.float32),
        in_specs=[pl.BlockSpec(memory_space=pltpu.VMEM)],
        out_specs=pl.BlockSpec(memory_space=pltpu.VMEM),
        scratch_shapes=[
            pltpu.VMEM((B, SQ_LOCAL, D_MODEL), jnp.bfloat16),
            pltpu.VMEM((N_DEV - 1, B, SQ_LOCAL, D_MODEL), jnp.bfloat16),
            pltpu.SemaphoreType.DMA((N_DEV - 1,)),
            pltpu.SemaphoreType.DMA((N_DEV - 1,)),
        ],
        compiler_params=_CompilerParams(collective_id=1),
    )(partial_bf)


def kernel(x, Wq, Wk, Wv, Wo):
    x_bf = x.astype(jnp.bfloat16)
    wq = Wq.astype(jnp.bfloat16)
    wk = Wk.astype(jnp.bfloat16)
    wv = Wv.astype(jnp.bfloat16)
    wo = Wo.astype(jnp.bfloat16)
    cos = jnp.asarray(_COS_NP)
    sin = jnp.asarray(_SIN_NP)
    rot = jnp.asarray(_ROT_NP).astype(jnp.bfloat16)

    x_full = _all_gather(x_bf)
    return _attn_rs(x_full, wq, wk, wv, wo, cos, sin, rot)


# device time: 268110 ns/iter; 1.6687x vs baseline; 1.1058x over previous
import jax
import jax.numpy as jnp
import numpy as np
from jax import lax
from jax.experimental import pallas as pl
from jax.experimental.pallas import tpu as pltpu

N_DEV = 4
B = 2
SQ_LOCAL = 512
SQ = N_DEV * SQ_LOCAL
D_MODEL = 1024
H_LOCAL = 8
D_HEAD = 128
QBLK = 512
SCALE = 0.08838834764831843

_CompilerParams = getattr(pltpu, "CompilerParams", None) or getattr(
    pltpu, "TPUCompilerParams"
)


def _rope_tables():
    inv = 1.0 / (10000.0 ** (np.arange(0, D_HEAD, 2) / D_HEAD))
    pos = np.arange(SQ)[:, None] * inv[None, :]
    cos = np.repeat(np.cos(pos), 2, axis=-1).astype(np.float32)
    sin = np.repeat(np.sin(pos), 2, axis=-1).astype(np.float32)
    rot = np.zeros((D_HEAD, D_HEAD), np.float32)
    for k2 in range(0, D_HEAD, 2):
        rot[k2 + 1, k2] = -1.0
        rot[k2, k2 + 1] = 1.0
    return cos, sin, rot


_COS_NP, _SIN_NP, _ROT_NP = _rope_tables()


def _ag_body(x_ref, out_ref, buf_a, buf_b, buf_c1, buf_c2, send_sems, recv_sems):
    my = lax.axis_index("i")
    left = (my - 1) % N_DEV
    right = (my + 1) % N_DEV
    half = SQ_LOCAL // 2

    barrier = pltpu.get_barrier_semaphore()
    for nbr in (left, right):
        pl.semaphore_signal(
            barrier, inc=1, device_id=(nbr,), device_id_type=pl.DeviceIdType.MESH
        )
    pl.semaphore_wait(barrier, 2)

    out_ref[:, pl.ds(my * SQ_LOCAL, SQ_LOCAL), :] = x_ref[...]

    r0 = pltpu.make_async_remote_copy(
        src_ref=x_ref, dst_ref=buf_a,
        send_sem=send_sems.at[0], recv_sem=recv_sems.at[0],
        device_id=(right,), device_id_type=pl.DeviceIdType.MESH,
    )
    l0 = pltpu.make_async_remote_copy(
        src_ref=x_ref, dst_ref=buf_b,
        send_sem=send_sems.at[1], recv_sem=recv_sems.at[1],
        device_id=(left,), device_id_type=pl.DeviceIdType.MESH,
    )
    r0.start()
    l0.start()

    r1 = pltpu.make_async_remote_copy(
        src_ref=buf_a.at[:, 0:half, :], dst_ref=buf_c1,
        send_sem=send_sems.at[2], recv_sem=recv_sems.at[2],
        device_id=(right,), device_id_type=pl.DeviceIdType.MESH,
    )
    l1 = pltpu.make_async_remote_copy(
        src_ref=buf_b.at[:, half:SQ_LOCAL, :], dst_ref=buf_c2,
        send_sem=send_sems.at[3], recv_sem=recv_sems.at[3],
        device_id=(left,), device_id_type=pl.DeviceIdType.MESH,
    )
    r0.wait_recv()
    r1.start()
    l0.wait_recv()
    l1.start()

    out_ref[:, pl.ds(left * SQ_LOCAL, SQ_LOCAL), :] = buf_a[...]
    out_ref[:, pl.ds(right * SQ_LOCAL, SQ_LOCAL), :] = buf_b[...]

    far = (my + 2) % N_DEV
    r1.wait_recv()
    out_ref[:, pl.ds(far * SQ_LOCAL, half), :] = buf_c1[...]
    l1.wait_recv()
    out_ref[:, pl.ds(far * SQ_LOCAL + half, half), :] = buf_c2[...]

    for rdma in (r0, l0, r1, l1):
        rdma.wait_send()


def _all_gather(x_bf):
    return pl.pallas_call(
        _ag_body,
        out_shape=jax.ShapeDtypeStruct((B, SQ, D_MODEL), jnp.bfloat16),
        in_specs=[pl.BlockSpec(memory_space=pltpu.VMEM)],
        out_specs=pl.BlockSpec(memory_space=pltpu.VMEM),
        scratch_shapes=[
            pltpu.VMEM((B, SQ_LOCAL, D_MODEL), jnp.bfloat16),
            pltpu.VMEM((B, SQ_LOCAL, D_MODEL), jnp.bfloat16),
            pltpu.VMEM((B, SQ_LOCAL // 2, D_MODEL), jnp.bfloat16),
            pltpu.VMEM((B, SQ_LOCAL // 2, D_MODEL), jnp.bfloat16),
            pltpu.SemaphoreType.DMA((4,)),
            pltpu.SemaphoreType.DMA((4,)),
        ],
        compiler_params=_CompilerParams(collective_id=0),
    )(x_bf)


def _attn_body(x_ref, wq_ref, wk_ref, wv_ref, wo_ref, cos_ref, sin_ref,
               rot_ref, out_ref, qs_ref, ks_ref, vs_ref):
    b = pl.program_id(0)
    h = pl.program_id(1)

    @pl.when(h == 0)
    def _():
        xb = x_ref[b]
        cos = cos_ref[...]
        sin = sin_ref[...]
        rot = rot_ref[...]

        for w_ref, dst_ref, do_rope, scale in (
            (wq_ref, qs_ref, True, SCALE),
            (wk_ref, ks_ref, True, 1.0),
            (wv_ref, vs_ref, False, 1.0),
        ):
            for cc in range(0, D_MODEL, 512):
                t = jnp.dot(
                    xb, w_ref[:, cc:cc + 512],
                    preferred_element_type=jnp.float32,
                )
                for j in range(512 // D_HEAD):
                    hh = cc // D_HEAD + j
                    tcol = t[:, j * D_HEAD:(j + 1) * D_HEAD]
                    if do_rope:
                        tr = jnp.dot(
                            tcol.astype(jnp.bfloat16), rot,
                            preferred_element_type=jnp.float32,
                        )
                        tcol = (tcol * cos + tr * sin) * scale
                    dst_ref[hh] = tcol.astype(jnp.bfloat16)

    q = qs_ref[h]
    k = ks_ref[h]
    v = vs_ref[h]
    wo_h = wo_ref[pl.ds(h * D_HEAD, D_HEAD), :]

    for i in range(SQ // QBLK):
        sl = slice(i * QBLK, (i + 1) * QBLK)
        s = lax.dot_general(
            q[sl], k, (((1,), (1,)), ((), ())),
            preferred_element_type=jnp.float32,
        )
        p = jnp.exp(s)
        denom = jnp.sum(p, axis=-1, keepdims=True)
        ctx = jnp.dot(p.astype(jnp.bfloat16), v,
                      preferred_element_type=jnp.float32)
        ctx = ctx * (1.0 / denom)
        delta = jnp.dot(ctx.astype(jnp.bfloat16), wo_h,
                        preferred_element_type=jnp.float32)

        @pl.when(h == 0)
        def _():
            out_ref[b, sl, :] = delta

        @pl.when(h != 0)
        def _():
            out_ref[b, sl, :] = out_ref[b, sl, :] + delta


def _attention(x_full, wq, wk, wv, wo, cos, sin, rot):
    full = pl.BlockSpec(memory_space=pltpu.VMEM)
    return pl.pallas_call(
        _attn_body,
        grid=(B, H_LOCAL),
        in_specs=[full] * 8,
        out_specs=full,
        out_shape=jax.ShapeDtypeStruct((B, SQ, D_MODEL), jnp.float32),
        scratch_shapes=[
            pltpu.VMEM((H_LOCAL, SQ, D_HEAD), jnp.bfloat16),
            pltpu.VMEM((H_LOCAL, SQ, D_HEAD), jnp.bfloat16),
            pltpu.VMEM((H_LOCAL, SQ, D_HEAD), jnp.bfloat16),
        ],
        compiler_params=_CompilerParams(vmem_limit_bytes=60 * 1024 * 1024),
    )(x_full, wq, wk, wv, wo, cos, sin, rot)


def _attn_rs_body(x_ref, wq_ref, wk_ref, wv_ref, wo_ref, cos_ref, sin_ref,
                  rot_ref, out_ref, ks_ref, vs_ref, qc_ref, send_ref, recv_ref,
                  send_sems, recv_sems):
    my = lax.axis_index("i")
    left = (my - 1) % N_DEV
    right = (my + 1) % N_DEV

    barrier = pltpu.get_barrier_semaphore()
    for nbr in (left, right):
        pl.semaphore_signal(
            barrier, inc=1, device_id=(nbr,), device_id_type=pl.DeviceIdType.MESH
        )
    pl.semaphore_wait(barrier, 2)

    rot = rot_ref[...]

    for b in range(B):
        xb = x_ref[b]
        for w_ref, dst_ref, do_rope in (
            (wk_ref, ks_ref, True),
            (wv_ref, vs_ref, False),
        ):
            for cc in range(0, D_MODEL, 512):
                t = jnp.dot(xb, w_ref[:, cc:cc + 512],
                            preferred_element_type=jnp.float32)
                for j in range(512 // D_HEAD):
                    hh = cc // D_HEAD + j
                    tcol = t[:, j * D_HEAD:(j + 1) * D_HEAD]
                    if do_rope:
                        tr = jnp.dot(tcol.astype(jnp.bfloat16), rot,
                                     preferred_element_type=jnp.float32)
                        tcol = tcol * cos_ref[...] + tr * sin_ref[...]
                    dst_ref[b, hh] = tcol.astype(jnp.bfloat16)

    rdmas = [
        pltpu.make_async_remote_copy(
            src_ref=send_ref,
            dst_ref=recv_ref.at[s],
            send_sem=send_sems.at[s],
            recv_sem=recv_sems.at[s],
            device_id=(right,),
            device_id_type=pl.DeviceIdType.MESH,
        )
        for s in range(N_DEV - 1)
    ]

    for s in range(N_DEV):
        c = (my + N_DEV - 1 - s) % N_DEV
        row = pl.ds(c * SQ_LOCAL, SQ_LOCAL)
        cos_c = cos_ref[row, :]
        sin_c = sin_ref[row, :]
        accs = []
        for b in range(B):
            xc = x_ref[b, row, :]
            for cc in range(0, D_MODEL, 512):
                t = jnp.dot(xc, wq_ref[:, cc:cc + 512],
                            preferred_element_type=jnp.float32)
                for j in range(512 // D_HEAD):
                    hh = cc // D_HEAD + j
                    tcol = t[:, j * D_HEAD:(j + 1) * D_HEAD]
                    tr = jnp.dot(tcol.astype(jnp.bfloat16), rot,
                                 preferred_element_type=jnp.float32)
                    qc_ref[hh] = (
                        (tcol * cos_c + tr * sin_c) * SCALE
                    ).astype(jnp.bfloat16)
            def head_body(h, acc, b=b):
                sres = lax.dot_general(
                    qc_ref[h], ks_ref[b, h], (((1,), (1,)), ((), ())),
                    preferred_element_type=jnp.float32,
                )
                p = jnp.exp(sres.astype(jnp.bfloat16))
                denom = jnp.sum(p.astype(jnp.float32), axis=-1, keepdims=True)
                ctx = jnp.dot(p, vs_ref[b, h],
                              preferred_element_type=jnp.float32)
                ctx = ctx * (1.0 / denom)
                return acc + jnp.dot(
                    ctx.astype(jnp.bfloat16),
                    wo_ref[pl.ds(h * D_HEAD, D_HEAD), :],
                    preferred_element_type=jnp.float32,
                )

            acc = lax.fori_loop(
                0, H_LOCAL, head_body,
                jnp.zeros((SQ_LOCAL, D_MODEL), jnp.float32),
            )
            accs.append(acc)

        if s == 0:
            for b in range(B):
                send_ref[b] = accs[b].astype(jnp.bfloat16)
            rdmas[0].start()
        elif s < N_DEV - 1:
            rdmas[s - 1].wait_recv()
            rdmas[s - 1].wait_send()
            for b in range(B):
                send_ref[b] = (
                    accs[b] + recv_ref[s - 1, b].astype(jnp.float32)
                ).astype(jnp.bfloat16)
            rdmas[s].start()
        else:
            rdmas[s - 1].wait_recv()
            for b in range(B):
                out_ref[b] = accs[b] + recv_ref[s - 1, b].astype(jnp.float32)
            rdmas[s - 1].wait_send()


def _attn_rs(x_full, wq, wk, wv, wo, cos, sin, rot):
    full = pl.BlockSpec(memory_space=pltpu.VMEM)
    return pl.pallas_call(
        _attn_rs_body,
        out_shape=jax.ShapeDtypeStruct((B, SQ_LOCAL, D_MODEL), jnp.float32),
        in_specs=[full] * 8,
        out_specs=full,
        scratch_shapes=[
            pltpu.VMEM((B, H_LOCAL, SQ, D_HEAD), jnp.bfloat16),
            pltpu.VMEM((B, H_LOCAL, SQ, D_HEAD), jnp.bfloat16),
            pltpu.VMEM((H_LOCAL, SQ_LOCAL, D_HEAD), jnp.bfloat16),
            pltpu.VMEM((B, SQ_LOCAL, D_MODEL), jnp.bfloat16),
            pltpu.VMEM((N_DEV - 1, B, SQ_LOCAL, D_MODEL), jnp.bfloat16),
            pltpu.SemaphoreType.DMA((N_DEV - 1,)),
            pltpu.SemaphoreType.DMA((N_DEV - 1,)),
        ],
        compiler_params=_CompilerParams(
            collective_id=1, vmem_limit_bytes=63 * 1024 * 1024
        ),
    )(x_full, wq, wk, wv, wo, cos, sin, rot)


def _rs_body(p_ref, out_ref, send_buf, recv_buf, send_sems, recv_sems):
    my = lax.axis_index("i")
    left = (my - 1) % N_DEV
    right = (my + 1) % N_DEV

    barrier = pltpu.get_barrier_semaphore()
    for nbr in (left, right):
        pl.semaphore_signal(
            barrier, inc=1, device_id=(nbr,), device_id_type=pl.DeviceIdType.MESH
        )
    pl.semaphore_wait(barrier, 2)

    for s in range(N_DEV - 1):
        c_send = (my + N_DEV - 1 - s) % N_DEV
        chunk = p_ref[:, pl.ds(c_send * SQ_LOCAL, SQ_LOCAL), :]
        if s == 0:
            send_buf[...] = chunk
        else:
            send_buf[...] = chunk + recv_buf[s - 1]
        rdma = pltpu.make_async_remote_copy(
            src_ref=send_buf,
            dst_ref=recv_buf.at[s],
            send_sem=send_sems.at[s],
            recv_sem=recv_sems.at[s],
            device_id=(right,),
            device_id_type=pl.DeviceIdType.MESH,
        )
        rdma.start()
        rdma.wait()

    mine = p_ref[:, pl.ds(my * SQ_LOCAL, SQ_LOCAL), :].astype(jnp.float32)
    out_ref[...] = mine + recv_buf[N_DEV - 2].astype(jnp.float32)


def _reduce_scatter(partial_bf):
    return pl.pallas_call(
        _rs_body,
        out_shape=jax.ShapeDtypeStruct((B, SQ_LOCAL, D_MODEL), jnp.float32),
        in_specs=[pl.BlockSpec(memory_space=pltpu.VMEM)],
        out_specs=pl.BlockSpec(memory_space=pltpu.VMEM),
        scratch_shapes=[
            pltpu.VMEM((B, SQ_LOCAL, D_MODEL), jnp.bfloat16),
            pltpu.VMEM((N_DEV - 1, B, SQ_LOCAL, D_MODEL), jnp.bfloat16),
            pltpu.SemaphoreType.DMA((N_DEV - 1,)),
            pltpu.SemaphoreType.DMA((N_DEV - 1,)),
        ],
        compiler_params=_CompilerParams(collective_id=1),
    )(partial_bf)


def kernel(x, Wq, Wk, Wv, Wo):
    x_bf = x.astype(jnp.bfloat16)
    wq = Wq.astype(jnp.bfloat16)
    wk = Wk.astype(jnp.bfloat16)
    wv = Wv.astype(jnp.bfloat16)
    wo = Wo.astype(jnp.bfloat16)
    cos = jnp.asarray(_COS_NP)
    sin = jnp.asarray(_SIN_NP)
    rot = jnp.asarray(_ROT_NP).astype(jnp.bfloat16)

    x_full = _all_gather(x_bf)
    return _attn_rs(x_full, wq, wk, wv, wo, cos, sin, rot)


# device time: 246082 ns/iter; 1.8181x vs baseline; 1.0895x over previous
import jax
import jax.numpy as jnp
import numpy as np
from jax import lax
from jax.experimental import pallas as pl
from jax.experimental.pallas import tpu as pltpu

N_DEV = 4
B = 2
SQ_LOCAL = 512
SQ = N_DEV * SQ_LOCAL
D_MODEL = 1024
H_LOCAL = 8
D_HEAD = 128
QBLK = 512
SCALE = 0.08838834764831843

_CompilerParams = getattr(pltpu, "CompilerParams", None) or getattr(
    pltpu, "TPUCompilerParams"
)


def _rope_tables():
    inv = 1.0 / (10000.0 ** (np.arange(0, D_HEAD, 2) / D_HEAD))
    pos = np.arange(SQ)[:, None] * inv[None, :]
    cos = np.repeat(np.cos(pos), 2, axis=-1).astype(np.float32)
    sin = np.repeat(np.sin(pos), 2, axis=-1).astype(np.float32)
    rot = np.zeros((D_HEAD, D_HEAD), np.float32)
    for k2 in range(0, D_HEAD, 2):
        rot[k2 + 1, k2] = -1.0
        rot[k2, k2 + 1] = 1.0
    return cos, sin, rot


_COS_NP, _SIN_NP, _ROT_NP = _rope_tables()


def _ag_body(x_ref, out_ref, buf_a, buf_b, buf_c1, buf_c2, send_sems, recv_sems):
    my = lax.axis_index("i")
    left = (my - 1) % N_DEV
    right = (my + 1) % N_DEV
    half = SQ_LOCAL // 2

    barrier = pltpu.get_barrier_semaphore()
    for nbr in (left, right):
        pl.semaphore_signal(
            barrier, inc=1, device_id=(nbr,), device_id_type=pl.DeviceIdType.MESH
        )
    pl.semaphore_wait(barrier, 2)

    out_ref[:, pl.ds(my * SQ_LOCAL, SQ_LOCAL), :] = x_ref[...]

    r0 = pltpu.make_async_remote_copy(
        src_ref=x_ref, dst_ref=buf_a,
        send_sem=send_sems.at[0], recv_sem=recv_sems.at[0],
        device_id=(right,), device_id_type=pl.DeviceIdType.MESH,
    )
    l0 = pltpu.make_async_remote_copy(
        src_ref=x_ref, dst_ref=buf_b,
        send_sem=send_sems.at[1], recv_sem=recv_sems.at[1],
        device_id=(left,), device_id_type=pl.DeviceIdType.MESH,
    )
    r0.start()
    l0.start()

    r1 = pltpu.make_async_remote_copy(
        src_ref=buf_a.at[:, 0:half, :], dst_ref=buf_c1,
        send_sem=send_sems.at[2], recv_sem=recv_sems.at[2],
        device_id=(right,), device_id_type=pl.DeviceIdType.MESH,
    )
    l1 = pltpu.make_async_remote_copy(
        src_ref=buf_b.at[:, half:SQ_LOCAL, :], dst_ref=buf_c2,
        send_sem=send_sems.at[3], recv_sem=recv_sems.at[3],
        device_id=(left,), device_id_type=pl.DeviceIdType.MESH,
    )
    r0.wait_recv()
    r1.start()
    l0.wait_recv()
    l1.start()

    out_ref[:, pl.ds(left * SQ_LOCAL, SQ_LOCAL), :] = buf_a[...]
    out_ref[:, pl.ds(right * SQ_LOCAL, SQ_LOCAL), :] = buf_b[...]

    far = (my + 2) % N_DEV
    r1.wait_recv()
    out_ref[:, pl.ds(far * SQ_LOCAL, half), :] = buf_c1[...]
    l1.wait_recv()
    out_ref[:, pl.ds(far * SQ_LOCAL + half, half), :] = buf_c2[...]

    for rdma in (r0, l0, r1, l1):
        rdma.wait_send()


def _all_gather(x_bf):
    return pl.pallas_call(
        _ag_body,
        out_shape=jax.ShapeDtypeStruct((B, SQ, D_MODEL), jnp.bfloat16),
        in_specs=[pl.BlockSpec(memory_space=pltpu.VMEM)],
        out_specs=pl.BlockSpec(memory_space=pltpu.VMEM),
        scratch_shapes=[
            pltpu.VMEM((B, SQ_LOCAL, D_MODEL), jnp.bfloat16),
            pltpu.VMEM((B, SQ_LOCAL, D_MODEL), jnp.bfloat16),
            pltpu.VMEM((B, SQ_LOCAL // 2, D_MODEL), jnp.bfloat16),
            pltpu.VMEM((B, SQ_LOCAL // 2, D_MODEL), jnp.bfloat16),
            pltpu.SemaphoreType.DMA((4,)),
            pltpu.SemaphoreType.DMA((4,)),
        ],
        compiler_params=_CompilerParams(collective_id=0),
    )(x_bf)


def _attn_body(x_ref, wq_ref, wk_ref, wv_ref, wo_ref, cos_ref, sin_ref,
               rot_ref, out_ref, qs_ref, ks_ref, vs_ref):
    b = pl.program_id(0)
    h = pl.program_id(1)

    @pl.when(h == 0)
    def _():
        xb = x_ref[b]
        cos = cos_ref[...]
        sin = sin_ref[...]
        rot = rot_ref[...]

        for w_ref, dst_ref, do_rope, scale in (
            (wq_ref, qs_ref, True, SCALE),
            (wk_ref, ks_ref, True, 1.0),
            (wv_ref, vs_ref, False, 1.0),
        ):
            for cc in range(0, D_MODEL, 512):
                t = jnp.dot(
                    xb, w_ref[:, cc:cc + 512],
                    preferred_element_type=jnp.float32,
                )
                for j in range(512 // D_HEAD):
                    hh = cc // D_HEAD + j
                    tcol = t[:, j * D_HEAD:(j + 1) * D_HEAD]
                    if do_rope:
                        tr = jnp.dot(
                            tcol.astype(jnp.bfloat16), rot,
                            preferred_element_type=jnp.float32,
                        )
                        tcol = (tcol * cos + tr * sin) * scale
                    dst_ref[hh] = tcol.astype(jnp.bfloat16)

    q = qs_ref[h]
    k = ks_ref[h]
    v = vs_ref[h]
    wo_h = wo_ref[pl.ds(h * D_HEAD, D_HEAD), :]

    for i in range(SQ // QBLK):
        sl = slice(i * QBLK, (i + 1) * QBLK)
        s = lax.dot_general(
            q[sl], k, (((1,), (1,)), ((), ())),
            preferred_element_type=jnp.float32,
        )
        p = jnp.exp(s)
        denom = jnp.sum(p, axis=-1, keepdims=True)
        ctx = jnp.dot(p.astype(jnp.bfloat16), v,
                      preferred_element_type=jnp.float32)
        ctx = ctx * (1.0 / denom)
        delta = jnp.dot(ctx.astype(jnp.bfloat16), wo_h,
                        preferred_element_type=jnp.float32)

        @pl.when(h == 0)
        def _():
            out_ref[b, sl, :] = delta

        @pl.when(h != 0)
        def _():
            out_ref[b, sl, :] = out_ref[b, sl, :] + delta


def _attention(x_full, wq, wk, wv, wo, cos, sin, rot):
    full = pl.BlockSpec(memory_space=pltpu.VMEM)
    return pl.pallas_call(
        _attn_body,
        grid=(B, H_LOCAL),
        in_specs=[full] * 8,
        out_specs=full,
        out_shape=jax.ShapeDtypeStruct((B, SQ, D_MODEL), jnp.float32),
        scratch_shapes=[
            pltpu.VMEM((H_LOCAL, SQ, D_HEAD), jnp.bfloat16),
            pltpu.VMEM((H_LOCAL, SQ, D_HEAD), jnp.bfloat16),
            pltpu.VMEM((H_LOCAL, SQ, D_HEAD), jnp.bfloat16),
        ],
        compiler_params=_CompilerParams(vmem_limit_bytes=60 * 1024 * 1024),
    )(x_full, wq, wk, wv, wo, cos, sin, rot)


def _attn_rs_body(x_ref, wq_ref, wk_ref, wv_ref, wo_ref, cos_ref, sin_ref,
                  rot_ref, out_ref, ks_ref, vs_ref, qc_ref, send_ref, recv_ref,
                  send_sems, recv_sems):
    my = lax.axis_index("i")
    left = (my - 1) % N_DEV
    right = (my + 1) % N_DEV

    barrier = pltpu.get_barrier_semaphore()
    for nbr in (left, right):
        pl.semaphore_signal(
            barrier, inc=1, device_id=(nbr,), device_id_type=pl.DeviceIdType.MESH
        )
    pl.semaphore_wait(barrier, 2)

    rot = rot_ref[...]

    for b in range(B):
        xb = x_ref[b]
        for w_ref, dst_ref, do_rope in (
            (wk_ref, ks_ref, True),
            (wv_ref, vs_ref, False),
        ):
            for cc in range(0, D_MODEL, 512):
                t = jnp.dot(xb, w_ref[:, cc:cc + 512],
                            preferred_element_type=jnp.float32)
                for j in range(512 // D_HEAD):
                    hh = cc // D_HEAD + j
                    tcol = t[:, j * D_HEAD:(j + 1) * D_HEAD]
                    if do_rope:
                        tr = jnp.dot(tcol.astype(jnp.bfloat16), rot,
                                     preferred_element_type=jnp.float32)
                        tcol = tcol * cos_ref[...] + tr * sin_ref[...]
                    dst_ref[b, hh] = tcol.astype(jnp.bfloat16)

    rdmas = [
        pltpu.make_async_remote_copy(
            src_ref=send_ref,
            dst_ref=recv_ref.at[s],
            send_sem=send_sems.at[s],
            recv_sem=recv_sems.at[s],
            device_id=(right,),
            device_id_type=pl.DeviceIdType.MESH,
        )
        for s in range(N_DEV - 1)
    ]

    for s in range(N_DEV):
        c = (my + N_DEV - 1 - s) % N_DEV
        row = pl.ds(c * SQ_LOCAL, SQ_LOCAL)
        cos_c = cos_ref[row, :]
        sin_c = sin_ref[row, :]
        accs = []
        for b in range(B):
            xc = x_ref[b, row, :]
            for cc in range(0, D_MODEL, 512):
                t = jnp.dot(xc, wq_ref[:, cc:cc + 512],
                            preferred_element_type=jnp.float32)
                for j in range(512 // D_HEAD):
                    hh = cc // D_HEAD + j
                    tcol = t[:, j * D_HEAD:(j + 1) * D_HEAD]
                    tr = jnp.dot(tcol.astype(jnp.bfloat16), rot,
                                 preferred_element_type=jnp.float32)
                    qc_ref[hh] = (
                        (tcol * cos_c + tr * sin_c) * SCALE
                    ).astype(jnp.bfloat16)
            def head_body(h, acc, b=b):
                sres = lax.dot_general(
                    qc_ref[h], ks_ref[b, h], (((1,), (1,)), ((), ())),
                    preferred_element_type=jnp.float32,
                )
                p = jnp.exp(sres.astype(jnp.bfloat16))
                denom = jnp.sum(p.astype(jnp.float32), axis=-1, keepdims=True)
                ctx = jnp.dot(p, vs_ref[b, h],
                              preferred_element_type=jnp.float32)
                ctx = ctx * (1.0 / denom)
                return acc + jnp.dot(
                    ctx.astype(jnp.bfloat16),
                    wo_ref[pl.ds(h * D_HEAD, D_HEAD), :],
                    preferred_element_type=jnp.float32,
                )

            acc = lax.fori_loop(
                0, H_LOCAL, head_body,
                jnp.zeros((SQ_LOCAL, D_MODEL), jnp.float32),
            )
            accs.append(acc)

        if s == 0:
            for b in range(B):
                send_ref[b] = accs[b].astype(jnp.bfloat16)
            rdmas[0].start()
        elif s < N_DEV - 1:
            rdmas[s - 1].wait_recv()
            rdmas[s - 1].wait_send()
            for b in range(B):
                send_ref[b] = (
                    accs[b] + recv_ref[s - 1, b].astype(jnp.float32)
                ).astype(jnp.bfloat16)
            rdmas[s].start()
        else:
            rdmas[s - 1].wait_recv()
            for b in range(B):
                out_ref[b] = accs[b] + recv_ref[s - 1, b].astype(jnp.float32)
            rdmas[s - 1].wait_send()


def _attn_rs(x_full, wq, wk, wv, wo, cos, sin, rot):
    full = pl.BlockSpec(memory_space=pltpu.VMEM)
    return pl.pallas_call(
        _attn_rs_body,
        out_shape=jax.ShapeDtypeStruct((B, SQ_LOCAL, D_MODEL), jnp.float32),
        in_specs=[full] * 8,
        out_specs=full,
        scratch_shapes=[
            pltpu.VMEM((B, H_LOCAL, SQ, D_HEAD), jnp.bfloat16),
            pltpu.VMEM((B, H_LOCAL, SQ, D_HEAD), jnp.bfloat16),
            pltpu.VMEM((H_LOCAL, SQ_LOCAL, D_HEAD), jnp.bfloat16),
            pltpu.VMEM((B, SQ_LOCAL, D_MODEL), jnp.bfloat16),
            pltpu.VMEM((N_DEV - 1, B, SQ_LOCAL, D_MODEL), jnp.bfloat16),
            pltpu.SemaphoreType.DMA((N_DEV - 1,)),
            pltpu.SemaphoreType.DMA((N_DEV - 1,)),
        ],
        compiler_params=_CompilerParams(
            collective_id=1, vmem_limit_bytes=63 * 1024 * 1024
        ),
    )(x_full, wq, wk, wv, wo, cos, sin, rot)


def _rs_body(p_ref, out_ref, send_buf, recv_buf, send_sems, recv_sems):
    my = lax.axis_index("i")
    left = (my - 1) % N_DEV
    right = (my + 1) % N_DEV

    barrier = pltpu.get_barrier_semaphore()
    for nbr in (left, right):
        pl.semaphore_signal(
            barrier, inc=1, device_id=(nbr,), device_id_type=pl.DeviceIdType.MESH
        )
    pl.semaphore_wait(barrier, 2)

    for s in range(N_DEV - 1):
        c_send = (my + N_DEV - 1 - s) % N_DEV
        chunk = p_ref[:, pl.ds(c_send * SQ_LOCAL, SQ_LOCAL), :]
        if s == 0:
            send_buf[...] = chunk
        else:
            send_buf[...] = chunk + recv_buf[s - 1]
        rdma = pltpu.make_async_remote_copy(
            src_ref=send_buf,
            dst_ref=recv_buf.at[s],
            send_sem=send_sems.at[s],
            recv_sem=recv_sems.at[s],
            device_id=(right,),
            device_id_type=pl.DeviceIdType.MESH,
        )
        rdma.start()
        rdma.wait()

    mine = p_ref[:, pl.ds(my * SQ_LOCAL, SQ_LOCAL), :].astype(jnp.float32)
    out_ref[...] = mine + recv_buf[N_DEV - 2].astype(jnp.float32)


def _reduce_scatter(partial_bf):
    return pl.pallas_call(
        _rs_body,
        out_shape=jax.ShapeDtypeStruct((B, SQ_LOCAL, D_MODEL), jnp.float32),
        in_specs=[pl.BlockSpec(memory_space=pltpu.VMEM)],
        out_specs=pl.BlockSpec(memory_space=pltpu.VMEM),
        scratch_shapes=[
            pltpu.VMEM((B, SQ_LOCAL, D_MODEL), jnp.bfloat16),
            pltpu.VMEM((N_DEV - 1, B, SQ_LOCAL, D_MODEL), jnp.bfloat16),
            pltpu.SemaphoreType.DMA((N_DEV - 1,)),
            pltpu.SemaphoreType.DMA((N_DEV - 1,)),
        ],
        compiler_params=_CompilerParams(collective_id=1),
    )(partial_bf)


def _mega_body(x_ref, wq_ref, wk_ref, wv_ref, wo_ref, cos_ref, sin_ref,
               rot_ref, out_ref, xg_ref, ks_ref, vs_ref, qc_ref, send_ref,
               recv_ref, ag_send_sems, ag_recv_sems, rs_send_sems,
               rs_recv_sems):
    my = lax.axis_index("i")
    left = (my - 1) % N_DEV
    right = (my + 1) % N_DEV
    far = (my + 2) % N_DEV
    half = SQ_LOCAL // 2

    barrier = pltpu.get_barrier_semaphore()
    for nbr in (left, right):
        pl.semaphore_signal(
            barrier, inc=1, device_id=(nbr,), device_id_type=pl.DeviceIdType.MESH
        )
    pl.semaphore_wait(barrier, 2)

    rot = rot_ref[...]

    def project_kv(c):
        row = pl.ds(c * SQ_LOCAL, SQ_LOCAL)
        cos_c = cos_ref[row, :]
        sin_c = sin_ref[row, :]
        for b in range(B):
            xc = xg_ref[c, b]
            for w_ref, dst_ref, do_rope in (
                (wk_ref, ks_ref, True),
                (wv_ref, vs_ref, False),
            ):
                t = jnp.dot(xc, w_ref[...], preferred_element_type=jnp.float32)
                for hh in range(H_LOCAL):
                    tcol = t[:, hh * D_HEAD:(hh + 1) * D_HEAD]
                    if do_rope:
                        tr = jnp.dot(tcol.astype(jnp.bfloat16), rot,
                                     preferred_element_type=jnp.float32)
                        tcol = tcol * cos_c + tr * sin_c
                    dst_ref[b, hh, row, :] = tcol.astype(jnp.bfloat16)

    r0 = pltpu.make_async_remote_copy(
        src_ref=x_ref, dst_ref=xg_ref.at[my],
        send_sem=ag_send_sems.at[0], recv_sem=ag_recv_sems.at[0],
        device_id=(right,), device_id_type=pl.DeviceIdType.MESH,
    )
    l0 = pltpu.make_async_remote_copy(
        src_ref=x_ref, dst_ref=xg_ref.at[my],
        send_sem=ag_send_sems.at[1], recv_sem=ag_recv_sems.at[1],
        device_id=(left,), device_id_type=pl.DeviceIdType.MESH,
    )
    r0.start()
    l0.start()
    xg_ref[my] = x_ref[...]
    project_kv(my)

    r1 = pltpu.make_async_remote_copy(
        src_ref=xg_ref.at[left, :, 0:half, :],
        dst_ref=xg_ref.at[left, :, 0:half, :],
        send_sem=ag_send_sems.at[2], recv_sem=ag_recv_sems.at[2],
        device_id=(right,), device_id_type=pl.DeviceIdType.MESH,
    )
    l1 = pltpu.make_async_remote_copy(
        src_ref=xg_ref.at[right, :, half:SQ_LOCAL, :],
        dst_ref=xg_ref.at[right, :, half:SQ_LOCAL, :],
        send_sem=ag_send_sems.at[3], recv_sem=ag_recv_sems.at[3],
        device_id=(left,), device_id_type=pl.DeviceIdType.MESH,
    )
    r0.wait_recv()
    r1.start()
    l0.wait_recv()
    l1.start()
    project_kv(left)
    project_kv(right)
    r1.wait_recv()
    l1.wait_recv()
    project_kv(far)
    for rdma in (r0, l0, r1, l1):
        rdma.wait_send()

    rdmas = [
        pltpu.make_async_remote_copy(
            src_ref=send_ref,
            dst_ref=recv_ref.at[s],
            send_sem=rs_send_sems.at[s],
            recv_sem=rs_recv_sems.at[s],
            device_id=(right,),
            device_id_type=pl.DeviceIdType.MESH,
        )
        for s in range(N_DEV - 1)
    ]

    for s in range(N_DEV):
        c = (my + N_DEV - 1 - s) % N_DEV
        row = pl.ds(c * SQ_LOCAL, SQ_LOCAL)
        cos_c = cos_ref[row, :]
        sin_c = sin_ref[row, :]
        accs = []
        for b in range(B):
            xc = xg_ref[c, b]
            t = jnp.dot(xc, wq_ref[...], preferred_element_type=jnp.float32)
            for hh in range(H_LOCAL):
                tcol = t[:, hh * D_HEAD:(hh + 1) * D_HEAD]
                tr = jnp.dot(tcol.astype(jnp.bfloat16), rot,
                             preferred_element_type=jnp.float32)
                qc_ref[hh] = (
                    (tcol * cos_c + tr * sin_c) * SCALE
                ).astype(jnp.bfloat16)

            def head_body(h, acc, b=b):
                sres = lax.dot_general(
                    qc_ref[h], ks_ref[b, h], (((1,), (1,)), ((), ())),
                    preferred_element_type=jnp.float32,
                )
                p = jnp.exp(sres.astype(jnp.bfloat16))
                denom = jnp.sum(p.astype(jnp.float32), axis=-1, keepdims=True)
                ctx = jnp.dot(p, vs_ref[b, h],
                              preferred_element_type=jnp.float32)
                ctx = ctx * (1.0 / denom)
                return acc + jnp.dot(
                    ctx.astype(jnp.bfloat16),
                    wo_ref[pl.ds(h * D_HEAD, D_HEAD), :],
                    preferred_element_type=jnp.float32,
                )

            accs.append(lax.fori_loop(
                0, H_LOCAL, head_body,
                jnp.zeros((SQ_LOCAL, D_MODEL), jnp.float32),
            ))

        if s == 0:
            for b in range(B):
                send_ref[b] = accs[b].astype(jnp.bfloat16)
            rdmas[0].start()
        elif s < N_DEV - 1:
            rdmas[s - 1].wait_recv()
            rdmas[s - 1].wait_send()
            for b in range(B):
                send_ref[b] = (
                    accs[b] + recv_ref[s - 1, b].astype(jnp.float32)
                ).astype(jnp.bfloat16)
            rdmas[s].start()
        else:
            rdmas[s - 1].wait_recv()
            for b in range(B):
                out_ref[b] = accs[b] + recv_ref[s - 1, b].astype(jnp.float32)
            rdmas[s - 1].wait_send()


def _mega(x_bf, wq, wk, wv, wo, cos, sin, rot):
    full = pl.BlockSpec(memory_space=pltpu.VMEM)
    return pl.pallas_call(
        _mega_body,
        out_shape=jax.ShapeDtypeStruct((B, SQ_LOCAL, D_MODEL), jnp.float32),
        in_specs=[full] * 8,
        out_specs=full,
        scratch_shapes=[
            pltpu.VMEM((N_DEV, B, SQ_LOCAL, D_MODEL), jnp.bfloat16),
            pltpu.VMEM((B, H_LOCAL, SQ, D_HEAD), jnp.bfloat16),
            pltpu.VMEM((B, H_LOCAL, SQ, D_HEAD), jnp.bfloat16),
            pltpu.VMEM((H_LOCAL, SQ_LOCAL, D_HEAD), jnp.bfloat16),
            pltpu.VMEM((B, SQ_LOCAL, D_MODEL), jnp.bfloat16),
            pltpu.VMEM((N_DEV - 1, B, SQ_LOCAL, D_MODEL), jnp.bfloat16),
            pltpu.SemaphoreType.DMA((4,)),
            pltpu.SemaphoreType.DMA((4,)),
            pltpu.SemaphoreType.DMA((N_DEV - 1,)),
            pltpu.SemaphoreType.DMA((N_DEV - 1,)),
        ],
        compiler_params=_CompilerParams(
            collective_id=0, vmem_limit_bytes=63 * 1024 * 1024
        ),
    )(x_bf, wq, wk, wv, wo, cos, sin, rot)


def kernel(x, Wq, Wk, Wv, Wo):
    x_bf = x.astype(jnp.bfloat16)
    wq = Wq.astype(jnp.bfloat16)
    wk = Wk.astype(jnp.bfloat16)
    wv = Wv.astype(jnp.bfloat16)
    wo = Wo.astype(jnp.bfloat16)
    cos = jnp.asarray(_COS_NP)
    sin = jnp.asarray(_SIN_NP)
    rot = jnp.asarray(_ROT_NP).astype(jnp.bfloat16)

    return _mega(x_bf, wq, wk, wv, wo, cos, sin, rot)
